# baseline (speedup 1.0000x reference)
import functools

import numpy as np
import jax
import jax.numpy as jnp
from jax import lax
from jax.experimental import pallas as pl
from jax.experimental.pallas import tpu as pltpu
from jax.experimental.pallas import tpu_sc as plsc

_I1_VALS = [70273749298880, 38956906369248, 16316086777680, 83297495521792,
            191839786542528, 376992761456332, 221880851359940, 0,
            -16781096230092, -27847728347500, -98222995813580, 0,
            793685538262556]

_ROW = 127
_H = 1024
_D = 13
_PLANE = _H * _D
_CH = _PLANE // 2
_NSLOTS = 11


def _sc_body(a1, a2, a3, a4, a5, m, out, buf, obuf, mbuf):
    wid = lax.axis_index("s") * 2 + lax.axis_index("c")
    slot = wid // 2
    h = wid % 2
    c0 = h * _CH

    pltpu.sync_copy(m, mbuf)

    for k, ref in enumerate((a1, a2, a3, a4)):
        @pl.when(slot == k)
        def _(ref=ref):
            pltpu.sync_copy(ref.at[_ROW, pl.ds(c0, _CH)], buf)

    for k in range(7):
        @pl.when(slot == 4 + k)
        def _(k=k):
            pltpu.sync_copy(a5.at[k * 128 + _ROW, pl.ds(c0, _CH)], buf)

    def body(g, carry):
        s = pl.ds(pl.multiple_of(g * 16, 16), 16)
        obuf[0, s] = buf[s] * mbuf[s]
        return carry

    lax.fori_loop(0, _CH // 16, body, 0)

    for k in range(_NSLOTS):
        @pl.when(slot == k)
        def _(k=k):
            pltpu.sync_copy(obuf, out.at[pl.ds(k, 1), pl.ds(c0, _CH)])


_sc_call = functools.partial(
    pl.kernel,
    _sc_body,
    out_type=jax.ShapeDtypeStruct((16, _PLANE), jnp.float32),
    mesh=plsc.VectorSubcoreMesh(core_axis_name="c", subcore_axis_name="s",
                                num_cores=2, num_subcores=16),
    compiler_params=pltpu.CompilerParams(needs_layout_passes=False,
                                         use_tc_tiling_on_sc=True),
    scratch_types=[
        pltpu.VMEM((_CH,), jnp.float32),
        pltpu.VMEM((1, _CH), jnp.float32),
        pltpu.VMEM((_CH,), jnp.float32),
    ],
)()


def kernel(x1, x2, x3, x4, x5, size):
    del size
    m = jnp.asarray(np.tile(np.asarray(_I1_VALS, dtype=np.float32),
                            _CH // _D))
    out = _sc_call(
        x1.reshape(128, _PLANE), x2.reshape(128, _PLANE),
        x3.reshape(128, _PLANE), x4.reshape(128, _PLANE),
        x5.reshape(7 * 128, _PLANE), m)
    return out[:_NSLOTS].reshape(_NSLOTS, 1, _H, _D)


# speedup vs baseline: 1.0155x; 1.0155x over previous
import functools

import numpy as np
import jax
import jax.numpy as jnp
from jax import lax
from jax.experimental import pallas as pl
from jax.experimental.pallas import tpu as pltpu
from jax.experimental.pallas import tpu_sc as plsc

_I1_VALS = [70273749298880, 38956906369248, 16316086777680, 83297495521792,
            191839786542528, 376992761456332, 221880851359940, 0,
            -16781096230092, -27847728347500, -98222995813580, 0,
            793685538262556]

_ROW = 127
_H = 1024
_D = 13
_PLANE = _H * _D
_CH = _PLANE // 2
_NSLOTS = 11


def _sc_body(a1, a2, a3, a4, a5, out, buf):
    wid = lax.axis_index("s") * 2 + lax.axis_index("c")
    slot = wid // 2
    h = wid % 2
    c0 = h * _CH

    for k, ref in enumerate((a1, a2, a3, a4)):
        @pl.when(slot == k)
        def _(ref=ref):
            pltpu.sync_copy(ref.at[pl.ds(_ROW, 1), pl.ds(c0, _CH)], buf)

    for k in range(7):
        @pl.when(slot == 4 + k)
        def _(k=k):
            pltpu.sync_copy(a5.at[pl.ds(k * 128 + _ROW, 1), pl.ds(c0, _CH)],
                            buf)

    for k in range(_NSLOTS):
        @pl.when(slot == k)
        def _(k=k):
            pltpu.sync_copy(buf, out.at[pl.ds(k, 1), pl.ds(c0, _CH)])


_sc_gather = functools.partial(
    pl.kernel,
    _sc_body,
    out_type=jax.ShapeDtypeStruct((16, _PLANE), jnp.float32),
    mesh=plsc.VectorSubcoreMesh(core_axis_name="c", subcore_axis_name="s",
                                num_cores=2, num_subcores=16),
    compiler_params=pltpu.CompilerParams(needs_layout_passes=False,
                                         use_tc_tiling_on_sc=True),
    scratch_types=[
        pltpu.VMEM((1, _CH), jnp.float32),
    ],
)()


def _tc_mul_body(g, m, out):
    out[...] = g[...] * m[...]


def kernel(x1, x2, x3, x4, x5, size):
    del size
    g = _sc_gather(
        x1.reshape(128, _PLANE), x2.reshape(128, _PLANE),
        x3.reshape(128, _PLANE), x4.reshape(128, _PLANE),
        x5.reshape(7 * 128, _PLANE))
    m = jnp.asarray(np.tile(np.asarray(_I1_VALS, dtype=np.float32),
                            _H).reshape(1, _PLANE))
    out = pl.pallas_call(
        _tc_mul_body,
        grid=(1,),
        out_shape=jax.ShapeDtypeStruct((16, _PLANE), jnp.float32),
        in_specs=[
            pl.BlockSpec((16, _PLANE), lambda i: (0, 0)),
            pl.BlockSpec((1, _PLANE), lambda i: (0, 0)),
        ],
        out_specs=pl.BlockSpec((16, _PLANE), lambda i: (0, 0)),
    )(g, m)
    return out[:_NSLOTS].reshape(_NSLOTS, 1, _H, _D)


# speedup vs baseline: 1.0167x; 1.0012x over previous
import functools

import numpy as np
import jax
import jax.numpy as jnp
from jax import lax
from jax.experimental import pallas as pl
from jax.experimental.pallas import tpu as pltpu
from jax.experimental.pallas import tpu_sc as plsc

_I1_VALS = [70273749298880, 38956906369248, 16316086777680, 83297495521792,
            191839786542528, 376992761456332, 221880851359940, 0,
            -16781096230092, -27847728347500, -98222995813580, 0,
            793685538262556]

_ROW = 127
_H = 1024
_D = 13
_PLANE = _H * _D
_CH = _PLANE // 2
_NSLOTS = 11


def _sc_body(a1, a2, a3, a4, a5, out, buf):
    wid = lax.axis_index("s") * 2 + lax.axis_index("c")
    slot = wid // 2
    h = wid % 2
    c0 = h * _CH

    for k, ref in enumerate((a1, a2, a3, a4)):
        @pl.when(slot == k)
        def _(ref=ref):
            pltpu.sync_copy(ref.at[pl.ds(_ROW, 1), pl.ds(c0, _CH)], buf)

    for k in range(7):
        @pl.when(slot == 4 + k)
        def _(k=k):
            pltpu.sync_copy(a5.at[pl.ds(k * 128 + _ROW, 1), pl.ds(c0, _CH)],
                            buf)

    for k in range(_NSLOTS):
        @pl.when(slot == k)
        def _(k=k):
            pltpu.sync_copy(buf, out.at[pl.ds(k, 1), pl.ds(c0, _CH)])


_sc_gather = functools.partial(
    pl.kernel,
    _sc_body,
    out_type=jax.ShapeDtypeStruct((16, _PLANE), jnp.float32),
    mesh=plsc.VectorSubcoreMesh(core_axis_name="c", subcore_axis_name="s",
                                num_cores=2, num_subcores=16),
    compiler_params=pltpu.CompilerParams(needs_layout_passes=False,
                                         use_tc_tiling_on_sc=True),
    scratch_types=[
        pltpu.VMEM((1, _CH), jnp.float32),
    ],
)()


def _tc_mul_body(g, m, out):
    out[...] = g[0:_NSLOTS, :] * m[...]


def kernel(x1, x2, x3, x4, x5, size):
    del size
    g = _sc_gather(
        x1.reshape(128, _PLANE), x2.reshape(128, _PLANE),
        x3.reshape(128, _PLANE), x4.reshape(128, _PLANE),
        x5.reshape(7 * 128, _PLANE))
    m = jnp.asarray(np.tile(np.asarray(_I1_VALS, dtype=np.float32),
                            _H).reshape(1, _PLANE))
    out = pl.pallas_call(
        _tc_mul_body,
        grid=(1,),
        out_shape=jax.ShapeDtypeStruct((_NSLOTS, _PLANE), jnp.float32),
        in_specs=[
            pl.BlockSpec((16, _PLANE), lambda i: (0, 0)),
            pl.BlockSpec((1, _PLANE), lambda i: (0, 0)),
        ],
        out_specs=pl.BlockSpec((_NSLOTS, _PLANE), lambda i: (0, 0)),
    )(g, m)
    return out.reshape(_NSLOTS, 1, _H, _D)
